# baseline (speedup 1.0000x reference)
import functools

import jax
import jax.numpy as jnp
from jax import lax
from jax.experimental import pallas as pl
from jax.experimental.pallas import tpu as pltpu
from jax.experimental.pallas import tpu_sc as plsc

N_B = 10000
NNZ = 320000
B = 256
BH = B // 2
D = 768
RA_SCALE = 2.0

NCORE = 2
NSUB = 16
ROWS_PER_CORE = N_B // NCORE
ACC_ROWS = 5008
DUMMY = ROWS_PER_CORE
CH = 80
CPT = 256
PNNZ = NSUB * CPT * CH
SBC = 64
NSB = CPT // SBC
NRING = 4
ZR = 64
ROWS_PER_TILE = 312


def _h_body(w_ref, xt_ref, b_ref, o0_ref, o1_ref):
    h = (jnp.dot(w_ref[...], xt_ref[...], preferred_element_type=jnp.float32)
         + b_ref[...])
    o0_ref[...] = h[:, :BH]
    o1_ref[...] = h[:, BH:]


def _out_body(x_ref, w_ref, b_ref, a0_ref, a1_ref, o_ref):
    mm = lax.dot_general(
        x_ref[...], w_ref[...], (((1,), (1,)), ((), ())),
        preferred_element_type=jnp.float32,
    )
    aT = jnp.concatenate([a0_ref[...].T, a1_ref[...].T], axis=0)
    o_ref[...] = mm + b_ref[...] + RA_SCALE * aT


_sc_mesh = plsc.VectorSubcoreMesh(core_axis_name="c", subcore_axis_name="s")


@functools.partial(
    pl.kernel,
    out_type=jax.ShapeDtypeStruct((N_B, BH), jnp.float32),
    mesh=_sc_mesh,
    compiler_params=pltpu.CompilerParams(needs_layout_passes=False),
    scratch_types=[
        pltpu.VMEM((SBC, 1, CH), jnp.int32),
        pltpu.VMEM((SBC, 1, CH), jnp.int32),
        pltpu.VMEM((SBC * CH,), jnp.float32),
        pltpu.VMEM((SBC, 1, CH), jnp.int32),
        pltpu.VMEM((CH * 16,), jnp.float32),
        pltpu.VMEM((CH, BH), jnp.float32),
        pltpu.VMEM((CH, BH), jnp.float32),
        pltpu.VMEM((ZR, BH), jnp.float32),
        pltpu.VMEM_SHARED((ACC_ROWS, BH), jnp.float32),
        pltpu.SemaphoreType.DMA,
        pltpu.SemaphoreType.DMA,
    ],
)
def _sc_spmm(hT, rows2, cols2, vals2, out,
             rbuf, cbuf, vbuf, ibuf, vsplat, gbuf0, gbuf1, zbuf, acc,
             gsem0, gsem1):
    c = lax.axis_index("c")
    s = lax.axis_index("s")
    core_base = c * ROWS_PER_CORE

    def _zrow(r, carry):
        for t in range(BH // 16):
            zbuf[r, pl.ds(t * 16, 16)] = jnp.zeros((16,), jnp.float32)
        return carry

    lax.fori_loop(0, ZR, _zrow, 0)
    zoff = 0
    for zlen in (ZR, ZR, ZR, ZR, ROWS_PER_TILE - 4 * ZR):
        pltpu.sync_copy(
            zbuf.at[pl.ds(0, zlen)],
            acc.at[pl.ds(s * ROWS_PER_TILE + zoff, zlen)])
        zoff += zlen

    @pl.when(s == 0)
    def _ztail():
        pltpu.sync_copy(
            zbuf.at[pl.ds(0, ACC_ROWS - NSUB * ROWS_PER_TILE)],
            acc.at[pl.ds(NSUB * ROWS_PER_TILE,
                         ACC_ROWS - NSUB * ROWS_PER_TILE)])

    plsc.subcore_barrier()

    iota16 = lax.iota(jnp.int32, 16)

    def _scale_chunk(buf, j):
        jc = j * CH
        for g in range(CH // 16):
            v16 = vbuf[pl.ds(jc + g * 16, 16)]
            base = iota16 * 16 + g * 256
            for i in range(16):
                plsc.store_scatter(vsplat, [base + i], v16)

        def _srow(kk, carry):
            for k2 in range(2):
                k = kk * 2 + k2
                vv = vsplat[pl.ds(k * 16, 16)]
                for t in range(BH // 16):
                    buf[k, pl.ds(t * 16, 16)] = buf[k, pl.ds(t * 16, 16)] * vv
            return carry

        lax.fori_loop(0, CH // 2, _srow, 0)

    def _superblock(sb, carry):
        cb = s * CPT + sb * SBC
        pltpu.sync_copy(rows2.at[pl.ds(cb, SBC)], rbuf)
        pltpu.sync_copy(cols2.at[pl.ds(cb, SBC)], cbuf)
        pltpu.sync_copy(vals2.at[pl.ds(cb * CH, SBC * CH)], vbuf)

        def _idx(j, icarry):
            for v in range(CH // 16):
                r = rbuf[j, 0, pl.ds(v * 16, 16)]
                lr = r - core_base
                m = (lr >= 0) & (lr < ROWS_PER_CORE)
                ibuf[j, 0, pl.ds(v * 16, 16)] = jnp.where(m, lr, DUMMY)
            return icarry

        lax.fori_loop(0, SBC, _idx, 0)

        def _pair(jp, icarry):
            jo = jp * 2
            g0 = pltpu.async_copy(hT.at[cbuf.at[jo, 0]], gbuf0, gsem0)
            g1 = pltpu.async_copy(hT.at[cbuf.at[jo + 1, 0]], gbuf1, gsem1)
            g0.wait()
            _scale_chunk(gbuf0, jo)
            pltpu.sync_copy(gbuf0, acc.at[ibuf.at[jo, 0]], add=True)
            g1.wait()
            _scale_chunk(gbuf1, jo + 1)
            pltpu.sync_copy(gbuf1, acc.at[ibuf.at[jo + 1, 0]], add=True)
            return icarry

        lax.fori_loop(0, SBC // 2, _pair, 0)
        return carry

    lax.fori_loop(0, NSB, _superblock, 0)
    plsc.subcore_barrier()

    base = s * ROWS_PER_TILE
    pltpu.sync_copy(
        acc.at[pl.ds(base, ROWS_PER_TILE)],
        out.at[pl.ds(c * ROWS_PER_CORE + base, ROWS_PER_TILE)],
    )

    @pl.when(s == 0)
    def _wtail():
        pltpu.sync_copy(
            acc.at[pl.ds(NSUB * ROWS_PER_TILE,
                         ROWS_PER_CORE - NSUB * ROWS_PER_TILE)],
            out.at[pl.ds(c * ROWS_PER_CORE + NSUB * ROWS_PER_TILE,
                         ROWS_PER_CORE - NSUB * ROWS_PER_TILE)],
        )


def kernel(input_feature, adj_indices, adj_values, W1, b1, W_agg, b_agg):
    x_t = input_feature.T
    pad = PNNZ - NNZ
    rows2 = jnp.concatenate(
        [adj_indices[0], jnp.zeros((pad,), jnp.int32)]).reshape(PNNZ // CH, 1, CH)
    cols2 = jnp.concatenate(
        [adj_indices[1], jnp.zeros((pad,), jnp.int32)]).reshape(PNNZ // CH, 1, CH)
    vals2 = jnp.concatenate([adj_values, jnp.zeros((pad,), jnp.float32)])

    blk = 1000
    grid = N_B // blk

    hT0, hT1 = pl.pallas_call(
        _h_body,
        grid=(grid,),
        in_specs=[
            pl.BlockSpec((blk, D), lambda i: (i, 0)),
            pl.BlockSpec((D, B), lambda i: (0, 0)),
            pl.BlockSpec((blk, 1), lambda i: (i, 0)),
        ],
        out_specs=[
            pl.BlockSpec((blk, BH), lambda i: (i, 0)),
            pl.BlockSpec((blk, BH), lambda i: (i, 0)),
        ],
        out_shape=[
            jax.ShapeDtypeStruct((N_B, BH), jnp.float32),
            jax.ShapeDtypeStruct((N_B, BH), jnp.float32),
        ],
        compiler_params=pltpu.CompilerParams(
            dimension_semantics=("parallel",)),
    )(W_agg, x_t, b_agg.reshape(N_B, 1))

    agg0 = _sc_spmm(hT0, rows2, cols2, vals2)
    agg1 = _sc_spmm(hT1, rows2, cols2, vals2)

    cblk = 1280
    cgrid = (N_B + cblk - 1) // cblk
    out = pl.pallas_call(
        _out_body,
        grid=(cgrid,),
        in_specs=[
            pl.BlockSpec((B, D), lambda j: (0, 0)),
            pl.BlockSpec((cblk, D), lambda j: (j, 0)),
            pl.BlockSpec((1, cblk), lambda j: (0, j)),
            pl.BlockSpec((cblk, BH), lambda j: (j, 0)),
            pl.BlockSpec((cblk, BH), lambda j: (j, 0)),
        ],
        out_specs=pl.BlockSpec((B, cblk), lambda j: (0, j)),
        out_shape=jax.ShapeDtypeStruct((B, N_B), jnp.float32),
        compiler_params=pltpu.CompilerParams(
            dimension_semantics=("parallel",)),
    )(input_feature, W1, b1.reshape(1, N_B), agg0, agg1)
    return out


# speedup vs baseline: 1.0808x; 1.0808x over previous
import functools

import jax
import jax.numpy as jnp
from jax import lax
from jax.experimental import pallas as pl
from jax.experimental.pallas import tpu as pltpu
from jax.experimental.pallas import tpu_sc as plsc

N_B = 10000
NNZ = 320000
B = 256
BH = B // 2
D = 768
RA_SCALE = 2.0

NCORE = 2
NSUB = 16
ROWS_PER_CORE = N_B // NCORE
ACC_ROWS = 5008
DUMMY = ROWS_PER_CORE
CH = 80
CPT = 256
PNNZ = NSUB * CPT * CH
SBC = 64
NSB = CPT // SBC
NRING = 4
ZR = 64
ROWS_PER_TILE = 312


def _h_body(w_ref, xt_ref, b_ref, o0_ref, o1_ref):
    h = (jnp.dot(w_ref[...], xt_ref[...], preferred_element_type=jnp.float32)
         + b_ref[...])
    o0_ref[...] = h[:, :BH]
    o1_ref[...] = h[:, BH:]


def _out_body(x_ref, w_ref, b_ref, a0_ref, a1_ref, o_ref):
    mm = lax.dot_general(
        x_ref[...], w_ref[...], (((1,), (1,)), ((), ())),
        preferred_element_type=jnp.float32,
    )
    aT = jnp.concatenate([a0_ref[...].T, a1_ref[...].T], axis=0)
    o_ref[...] = mm + b_ref[...] + RA_SCALE * aT


_sc_mesh = plsc.VectorSubcoreMesh(core_axis_name="c", subcore_axis_name="s")


@functools.partial(
    pl.kernel,
    out_type=jax.ShapeDtypeStruct((N_B, BH), jnp.float32),
    mesh=_sc_mesh,
    compiler_params=pltpu.CompilerParams(needs_layout_passes=False),
    scratch_types=[
        pltpu.VMEM((SBC, 1, CH), jnp.int32),
        pltpu.VMEM((SBC, 1, CH), jnp.int32),
        pltpu.VMEM((SBC * CH,), jnp.float32),
        pltpu.VMEM((SBC, 1, CH), jnp.int32),
        pltpu.VMEM((CH, BH), jnp.float32),
        pltpu.VMEM((CH, BH), jnp.float32),
        pltpu.VMEM((ZR, BH), jnp.float32),
        pltpu.VMEM_SHARED((ACC_ROWS, BH), jnp.float32),
        pltpu.SemaphoreType.DMA,
        pltpu.SemaphoreType.DMA,
    ],
)
def _sc_spmm(hT, rows2, cols2, vals2, out,
             rbuf, cbuf, vbuf, ibuf, gbuf0, gbuf1, zbuf, acc,
             gsem0, gsem1):
    c = lax.axis_index("c")
    s = lax.axis_index("s")
    core_base = c * ROWS_PER_CORE

    def _zrow(r, carry):
        for t in range(BH // 16):
            zbuf[r, pl.ds(t * 16, 16)] = jnp.zeros((16,), jnp.float32)
        return carry

    lax.fori_loop(0, ZR, _zrow, 0)
    zoff = 0
    for zlen in (ZR, ZR, ZR, ZR, ROWS_PER_TILE - 4 * ZR):
        pltpu.sync_copy(
            zbuf.at[pl.ds(0, zlen)],
            acc.at[pl.ds(s * ROWS_PER_TILE + zoff, zlen)])
        zoff += zlen

    @pl.when(s == 0)
    def _ztail():
        pltpu.sync_copy(
            zbuf.at[pl.ds(0, ACC_ROWS - NSUB * ROWS_PER_TILE)],
            acc.at[pl.ds(NSUB * ROWS_PER_TILE,
                         ACC_ROWS - NSUB * ROWS_PER_TILE)])

    plsc.subcore_barrier()

    def _scale_chunk(buf, j):
        jc = j * CH

        def _sgrp(g, carry):
            v16 = vbuf[pl.ds(jc + g * 16, 16)]
            for jj in range(16):
                vv = jnp.take_along_axis(
                    v16, jnp.full((16,), jj, jnp.int32), axis=0)
                k = g * 16 + jj
                for t in range(BH // 16):
                    buf[k, pl.ds(t * 16, 16)] = buf[k, pl.ds(t * 16, 16)] * vv
            return carry

        lax.fori_loop(0, CH // 16, _sgrp, 0)

    def _superblock(sb, carry):
        cb = s * CPT + sb * SBC
        pltpu.sync_copy(rows2.at[pl.ds(cb, SBC)], rbuf)
        pltpu.sync_copy(cols2.at[pl.ds(cb, SBC)], cbuf)
        pltpu.sync_copy(vals2.at[pl.ds(cb * CH, SBC * CH)], vbuf)

        def _idx(j, icarry):
            for v in range(CH // 16):
                r = rbuf[j, 0, pl.ds(v * 16, 16)]
                lr = r - core_base
                m = (lr >= 0) & (lr < ROWS_PER_CORE)
                ibuf[j, 0, pl.ds(v * 16, 16)] = jnp.where(m, lr, DUMMY)
            return icarry

        lax.fori_loop(0, SBC, _idx, 0)

        def _pair(jp, icarry):
            jo = jp * 2
            g0 = pltpu.async_copy(hT.at[cbuf.at[jo, 0]], gbuf0, gsem0)
            g1 = pltpu.async_copy(hT.at[cbuf.at[jo + 1, 0]], gbuf1, gsem1)
            g0.wait()
            _scale_chunk(gbuf0, jo)
            pltpu.sync_copy(gbuf0, acc.at[ibuf.at[jo, 0]], add=True)
            g1.wait()
            _scale_chunk(gbuf1, jo + 1)
            pltpu.sync_copy(gbuf1, acc.at[ibuf.at[jo + 1, 0]], add=True)
            return icarry

        lax.fori_loop(0, SBC // 2, _pair, 0)
        return carry

    lax.fori_loop(0, NSB, _superblock, 0)
    plsc.subcore_barrier()

    base = s * ROWS_PER_TILE
    pltpu.sync_copy(
        acc.at[pl.ds(base, ROWS_PER_TILE)],
        out.at[pl.ds(c * ROWS_PER_CORE + base, ROWS_PER_TILE)],
    )

    @pl.when(s == 0)
    def _wtail():
        pltpu.sync_copy(
            acc.at[pl.ds(NSUB * ROWS_PER_TILE,
                         ROWS_PER_CORE - NSUB * ROWS_PER_TILE)],
            out.at[pl.ds(c * ROWS_PER_CORE + NSUB * ROWS_PER_TILE,
                         ROWS_PER_CORE - NSUB * ROWS_PER_TILE)],
        )


def kernel(input_feature, adj_indices, adj_values, W1, b1, W_agg, b_agg):
    x_t = input_feature.T
    pad = PNNZ - NNZ
    rows2 = jnp.concatenate(
        [adj_indices[0], jnp.zeros((pad,), jnp.int32)]).reshape(PNNZ // CH, 1, CH)
    cols2 = jnp.concatenate(
        [adj_indices[1], jnp.zeros((pad,), jnp.int32)]).reshape(PNNZ // CH, 1, CH)
    vals2 = jnp.concatenate([adj_values, jnp.zeros((pad,), jnp.float32)])

    blk = 1000
    grid = N_B // blk

    hT0, hT1 = pl.pallas_call(
        _h_body,
        grid=(grid,),
        in_specs=[
            pl.BlockSpec((blk, D), lambda i: (i, 0)),
            pl.BlockSpec((D, B), lambda i: (0, 0)),
            pl.BlockSpec((blk, 1), lambda i: (i, 0)),
        ],
        out_specs=[
            pl.BlockSpec((blk, BH), lambda i: (i, 0)),
            pl.BlockSpec((blk, BH), lambda i: (i, 0)),
        ],
        out_shape=[
            jax.ShapeDtypeStruct((N_B, BH), jnp.float32),
            jax.ShapeDtypeStruct((N_B, BH), jnp.float32),
        ],
        compiler_params=pltpu.CompilerParams(
            dimension_semantics=("parallel",)),
    )(W_agg, x_t, b_agg.reshape(N_B, 1))

    agg0 = _sc_spmm(hT0, rows2, cols2, vals2)
    agg1 = _sc_spmm(hT1, rows2, cols2, vals2)

    cblk = 1280
    cgrid = (N_B + cblk - 1) // cblk
    out = pl.pallas_call(
        _out_body,
        grid=(cgrid,),
        in_specs=[
            pl.BlockSpec((B, D), lambda j: (0, 0)),
            pl.BlockSpec((cblk, D), lambda j: (j, 0)),
            pl.BlockSpec((1, cblk), lambda j: (0, j)),
            pl.BlockSpec((cblk, BH), lambda j: (j, 0)),
            pl.BlockSpec((cblk, BH), lambda j: (j, 0)),
        ],
        out_specs=pl.BlockSpec((B, cblk), lambda j: (0, j)),
        out_shape=jax.ShapeDtypeStruct((B, N_B), jnp.float32),
        compiler_params=pltpu.CompilerParams(
            dimension_semantics=("parallel",)),
    )(input_feature, W1, b1.reshape(1, N_B), agg0, agg1)
    return out


# speedup vs baseline: 2.2337x; 2.0666x over previous
import functools

import jax
import jax.numpy as jnp
from jax import lax
from jax.experimental import pallas as pl
from jax.experimental.pallas import tpu as pltpu
from jax.experimental.pallas import tpu_sc as plsc

N_B = 10000
NNZ = 320000
B = 256
BH = B // 2
D = 768
RA_SCALE = 2.0

NCORE = 2
NSUB = 16
ROWS_PER_CORE = N_B // NCORE
ACC_ROWS = 5008
DUMMY = ROWS_PER_CORE
CH = 80
CPT = NNZ // NSUB // CH
SBC = 50
NSB = CPT // SBC
ZR = 64
ROWS_PER_TILE = 312


def _h_body(w_ref, xt_ref, b_ref, o0_ref, o1_ref):
    h = (jnp.dot(w_ref[...], xt_ref[...], preferred_element_type=jnp.float32)
         + b_ref[...])
    o0_ref[...] = h[:, :BH]
    o1_ref[...] = h[:, BH:]


def _out_body(x_ref, w_ref, b_ref, a0_ref, a1_ref, o_ref):
    mm = lax.dot_general(
        x_ref[...], w_ref[...], (((1,), (1,)), ((), ())),
        preferred_element_type=jnp.float32,
    )
    aT = jnp.concatenate([a0_ref[...].T, a1_ref[...].T], axis=0)
    o_ref[...] = mm + b_ref[...] + RA_SCALE * aT


_sc_mesh = plsc.VectorSubcoreMesh(core_axis_name="c", subcore_axis_name="s")


@functools.partial(
    pl.kernel,
    out_type=jax.ShapeDtypeStruct((N_B, BH), jnp.float32),
    mesh=_sc_mesh,
    compiler_params=pltpu.CompilerParams(needs_layout_passes=False),
    scratch_types=[
        pltpu.VMEM((SBC, 1, CH), jnp.int32),
        pltpu.VMEM((SBC, 1, CH), jnp.int32),
        pltpu.VMEM((SBC * CH,), jnp.float32),
        pltpu.VMEM((SBC, 1, CH), jnp.int32),
        pltpu.VMEM((CH, BH), jnp.float32),
        pltpu.VMEM((CH, BH), jnp.float32),
        pltpu.VMEM((ZR, BH), jnp.float32),
        pltpu.VMEM_SHARED((ACC_ROWS, BH), jnp.float32),
        pltpu.SemaphoreType.DMA,
        pltpu.SemaphoreType.DMA,
    ],
)
def _sc_spmm(hT, rows2, cols2, vals2, out,
             rbuf, cbuf, vbuf, ibuf, gbuf0, gbuf1, zbuf, acc,
             gsem0, gsem1):
    c = lax.axis_index("c")
    s = lax.axis_index("s")
    core_base = c * ROWS_PER_CORE

    def _zrow(r, carry):
        for t in range(BH // 16):
            zbuf[r, pl.ds(t * 16, 16)] = jnp.zeros((16,), jnp.float32)
        return carry

    lax.fori_loop(0, ZR, _zrow, 0)
    zoff = 0
    for zlen in (ZR, ZR, ZR, ZR, ROWS_PER_TILE - 4 * ZR):
        pltpu.sync_copy(
            zbuf.at[pl.ds(0, zlen)],
            acc.at[pl.ds(s * ROWS_PER_TILE + zoff, zlen)])
        zoff += zlen

    @pl.when(s == 0)
    def _ztail():
        pltpu.sync_copy(
            zbuf.at[pl.ds(0, ACC_ROWS - NSUB * ROWS_PER_TILE)],
            acc.at[pl.ds(NSUB * ROWS_PER_TILE,
                         ACC_ROWS - NSUB * ROWS_PER_TILE)])

    plsc.subcore_barrier()

    def _scale_chunk(buf, j):
        jc = j * CH

        def _sgrp(g, carry):
            v16 = vbuf[pl.ds(jc + g * 16, 16)]
            for jj in range(16):
                vv = jnp.take_along_axis(
                    v16, jnp.full((16,), jj, jnp.int32), axis=0)
                k = g * 16 + jj
                for t in range(BH // 16):
                    buf[k, pl.ds(t * 16, 16)] = buf[k, pl.ds(t * 16, 16)] * vv
            return carry

        lax.fori_loop(0, CH // 16, _sgrp, 0)

    def _superblock(sb, carry):
        cb = s * CPT + sb * SBC
        pltpu.sync_copy(rows2.at[pl.ds(cb, SBC)], rbuf)
        pltpu.sync_copy(cols2.at[pl.ds(cb, SBC)], cbuf)
        pltpu.sync_copy(vals2.at[pl.ds(cb * CH, SBC * CH)], vbuf)

        def _idx(j, icarry):
            for v in range(CH // 16):
                r = rbuf[j, 0, pl.ds(v * 16, 16)]
                lr = r - core_base
                m = (lr >= 0) & (lr < ROWS_PER_CORE)
                ibuf[j, 0, pl.ds(v * 16, 16)] = jnp.where(m, lr, DUMMY)
            return icarry

        lax.fori_loop(0, SBC, _idx, 0)

        def _pair(jp, icarry):
            jo = jp * 2
            g0 = pltpu.async_copy(hT.at[cbuf.at[jo, 0]], gbuf0, gsem0)
            g1 = pltpu.async_copy(hT.at[cbuf.at[jo + 1, 0]], gbuf1, gsem1)
            g0.wait()
            _scale_chunk(gbuf0, jo)
            pltpu.sync_copy(gbuf0, acc.at[ibuf.at[jo, 0]], add=True)
            g1.wait()
            _scale_chunk(gbuf1, jo + 1)
            pltpu.sync_copy(gbuf1, acc.at[ibuf.at[jo + 1, 0]], add=True)
            return icarry

        lax.fori_loop(0, SBC // 2, _pair, 0)
        return carry

    lax.fori_loop(0, NSB, _superblock, 0)
    plsc.subcore_barrier()

    base = s * ROWS_PER_TILE
    pltpu.sync_copy(
        acc.at[pl.ds(base, ROWS_PER_TILE)],
        out.at[pl.ds(c * ROWS_PER_CORE + base, ROWS_PER_TILE)],
    )

    @pl.when(s == 0)
    def _wtail():
        pltpu.sync_copy(
            acc.at[pl.ds(NSUB * ROWS_PER_TILE,
                         ROWS_PER_CORE - NSUB * ROWS_PER_TILE)],
            out.at[pl.ds(c * ROWS_PER_CORE + NSUB * ROWS_PER_TILE,
                         ROWS_PER_CORE - NSUB * ROWS_PER_TILE)],
        )


def kernel(input_feature, adj_indices, adj_values, W1, b1, W_agg, b_agg):
    x_t = input_feature.T
    rows2 = adj_indices[0].reshape(NNZ // CH, 1, CH)
    cols2 = adj_indices[1].reshape(NNZ // CH, 1, CH)
    vals2 = adj_values

    blk = 1000
    grid = N_B // blk

    hT0, hT1 = pl.pallas_call(
        _h_body,
        grid=(grid,),
        in_specs=[
            pl.BlockSpec((blk, D), lambda i: (i, 0)),
            pl.BlockSpec((D, B), lambda i: (0, 0)),
            pl.BlockSpec((blk, 1), lambda i: (i, 0)),
        ],
        out_specs=[
            pl.BlockSpec((blk, BH), lambda i: (i, 0)),
            pl.BlockSpec((blk, BH), lambda i: (i, 0)),
        ],
        out_shape=[
            jax.ShapeDtypeStruct((N_B, BH), jnp.float32),
            jax.ShapeDtypeStruct((N_B, BH), jnp.float32),
        ],
        compiler_params=pltpu.CompilerParams(
            dimension_semantics=("parallel",)),
    )(W_agg, x_t, b_agg.reshape(N_B, 1))

    agg0 = _sc_spmm(hT0, rows2, cols2, vals2)
    agg1 = _sc_spmm(hT1, rows2, cols2, vals2)

    cblk = 1280
    cgrid = (N_B + cblk - 1) // cblk
    out = pl.pallas_call(
        _out_body,
        grid=(cgrid,),
        in_specs=[
            pl.BlockSpec((B, D), lambda j: (0, 0)),
            pl.BlockSpec((cblk, D), lambda j: (j, 0)),
            pl.BlockSpec((1, cblk), lambda j: (0, j)),
            pl.BlockSpec((cblk, BH), lambda j: (j, 0)),
            pl.BlockSpec((cblk, BH), lambda j: (j, 0)),
        ],
        out_specs=pl.BlockSpec((B, cblk), lambda j: (0, j)),
        out_shape=jax.ShapeDtypeStruct((B, N_B), jnp.float32),
        compiler_params=pltpu.CompilerParams(
            dimension_semantics=("parallel",)),
    )(input_feature, W1, b1.reshape(1, N_B), agg0, agg1)
    return out


# speedup vs baseline: 2.4088x; 1.0784x over previous
import functools

import jax
import jax.numpy as jnp
from jax import lax
from jax.experimental import pallas as pl
from jax.experimental.pallas import tpu as pltpu
from jax.experimental.pallas import tpu_sc as plsc

N_B = 10000
NNZ = 320000
B = 256
BH = B // 2
D = 768
RA_SCALE = 2.0

NCORE = 2
NSUB = 16
ROWS_PER_CORE = N_B // NCORE
ACC_ROWS = 5008
DUMMY = ROWS_PER_CORE
CH = 80
CPT = NNZ // NSUB // CH
SBC = 50
NSB = CPT // SBC
ZR = 64
ROWS_PER_TILE = 312


def _h_body(w_ref, xt_ref, b_ref, o0_ref, o1_ref):
    h = (jnp.dot(w_ref[...], xt_ref[...], preferred_element_type=jnp.float32)
         + b_ref[...])
    o0_ref[...] = h[:, :BH]
    o1_ref[...] = h[:, BH:]


def _out_body(x_ref, w_ref, b_ref, a0_ref, a1_ref, o_ref):
    mm = lax.dot_general(
        x_ref[...], w_ref[...], (((1,), (1,)), ((), ())),
        preferred_element_type=jnp.float32,
    )
    aT = jnp.concatenate([a0_ref[...].T, a1_ref[...].T], axis=0)
    o_ref[...] = mm + b_ref[...] + RA_SCALE * aT


_sc_mesh = plsc.VectorSubcoreMesh(core_axis_name="c", subcore_axis_name="s")


@functools.partial(
    pl.kernel,
    out_type=jax.ShapeDtypeStruct((N_B, BH), jnp.float32),
    mesh=_sc_mesh,
    compiler_params=pltpu.CompilerParams(needs_layout_passes=False),
    scratch_types=[
        pltpu.VMEM((SBC, 1, CH), jnp.int32),
        pltpu.VMEM((SBC, 1, CH), jnp.int32),
        pltpu.VMEM((SBC * CH,), jnp.float32),
        pltpu.VMEM((SBC, 1, CH), jnp.int32),
        pltpu.VMEM((CH, BH), jnp.float32),
        pltpu.VMEM((CH, BH), jnp.float32),
        pltpu.VMEM((ZR, BH), jnp.float32),
        pltpu.VMEM_SHARED((ACC_ROWS, BH), jnp.float32),
        pltpu.SemaphoreType.DMA,
        pltpu.SemaphoreType.DMA,
        pltpu.SemaphoreType.DMA,
        pltpu.SemaphoreType.DMA,
    ],
)
def _sc_spmm(hT, rows2, cols2, vals2, out,
             rbuf, cbuf, vbuf, ibuf, gbuf0, gbuf1, zbuf, acc,
             gsem0, gsem1, ssem0, ssem1):
    c = lax.axis_index("c")
    s = lax.axis_index("s")
    core_base = c * ROWS_PER_CORE

    def _zrow(r, carry):
        for t in range(BH // 16):
            zbuf[r, pl.ds(t * 16, 16)] = jnp.zeros((16,), jnp.float32)
        return carry

    lax.fori_loop(0, ZR, _zrow, 0)
    zoff = 0
    for zlen in (ZR, ZR, ZR, ZR, ROWS_PER_TILE - 4 * ZR):
        pltpu.sync_copy(
            zbuf.at[pl.ds(0, zlen)],
            acc.at[pl.ds(s * ROWS_PER_TILE + zoff, zlen)])
        zoff += zlen

    @pl.when(s == 0)
    def _ztail():
        pltpu.sync_copy(
            zbuf.at[pl.ds(0, ACC_ROWS - NSUB * ROWS_PER_TILE)],
            acc.at[pl.ds(NSUB * ROWS_PER_TILE,
                         ACC_ROWS - NSUB * ROWS_PER_TILE)])

    plsc.subcore_barrier()

    def _scale_chunk(buf, j):
        jc = j * CH

        def _sgrp(g, carry):
            v16 = vbuf[pl.ds(jc + g * 16, 16)]
            for jj in range(16):
                vv = jnp.take_along_axis(
                    v16, jnp.full((16,), jj, jnp.int32), axis=0)
                k = g * 16 + jj
                for t in range(BH // 16):
                    buf[k, pl.ds(t * 16, 16)] = buf[k, pl.ds(t * 16, 16)] * vv
            return carry

        lax.fori_loop(0, CH // 16, _sgrp, 0)

    def _superblock(sb, carry):
        cb = s * CPT + sb * SBC
        pltpu.sync_copy(rows2.at[pl.ds(cb, SBC)], rbuf)
        pltpu.sync_copy(cols2.at[pl.ds(cb, SBC)], cbuf)
        pltpu.sync_copy(vals2.at[pl.ds(cb * CH, SBC * CH)], vbuf)

        def _idx(j, icarry):
            for v in range(CH // 16):
                r = rbuf[j, 0, pl.ds(v * 16, 16)]
                lr = r - core_base
                m = (lr >= 0) & (lr < ROWS_PER_CORE)
                ibuf[j, 0, pl.ds(v * 16, 16)] = jnp.where(m, lr, DUMMY)
            return icarry

        lax.fori_loop(0, SBC, _idx, 0)

        def _pair(jp, icarry):
            jo = jp * 2

            @pl.when((jp > 0) | (sb > 0))
            def _drain():
                pltpu.make_async_copy(
                    gbuf0, acc.at[ibuf.at[jo, 0]], ssem0).wait()
                pltpu.make_async_copy(
                    gbuf1, acc.at[ibuf.at[jo, 0]], ssem1).wait()

            g0 = pltpu.async_copy(hT.at[cbuf.at[jo, 0]], gbuf0, gsem0)
            g1 = pltpu.async_copy(hT.at[cbuf.at[jo + 1, 0]], gbuf1, gsem1)
            g0.wait()
            _scale_chunk(gbuf0, jo)
            pltpu.async_copy(gbuf0, acc.at[ibuf.at[jo, 0]], ssem0, add=True)
            g1.wait()
            _scale_chunk(gbuf1, jo + 1)
            pltpu.async_copy(gbuf1, acc.at[ibuf.at[jo + 1, 0]], ssem1, add=True)
            return icarry

        lax.fori_loop(0, SBC // 2, _pair, 0)
        return carry

    lax.fori_loop(0, NSB, _superblock, 0)

    pltpu.make_async_copy(gbuf0, acc.at[ibuf.at[0, 0]], ssem0).wait()
    pltpu.make_async_copy(gbuf1, acc.at[ibuf.at[0, 0]], ssem1).wait()
    plsc.subcore_barrier()

    base = s * ROWS_PER_TILE
    pltpu.sync_copy(
        acc.at[pl.ds(base, ROWS_PER_TILE)],
        out.at[pl.ds(c * ROWS_PER_CORE + base, ROWS_PER_TILE)],
    )

    @pl.when(s == 0)
    def _wtail():
        pltpu.sync_copy(
            acc.at[pl.ds(NSUB * ROWS_PER_TILE,
                         ROWS_PER_CORE - NSUB * ROWS_PER_TILE)],
            out.at[pl.ds(c * ROWS_PER_CORE + NSUB * ROWS_PER_TILE,
                         ROWS_PER_CORE - NSUB * ROWS_PER_TILE)],
        )


def kernel(input_feature, adj_indices, adj_values, W1, b1, W_agg, b_agg):
    x_t = input_feature.T
    rows2 = adj_indices[0].reshape(NNZ // CH, 1, CH)
    cols2 = adj_indices[1].reshape(NNZ // CH, 1, CH)
    vals2 = adj_values

    blk = 1000
    grid = N_B // blk

    hT0, hT1 = pl.pallas_call(
        _h_body,
        grid=(grid,),
        in_specs=[
            pl.BlockSpec((blk, D), lambda i: (i, 0)),
            pl.BlockSpec((D, B), lambda i: (0, 0)),
            pl.BlockSpec((blk, 1), lambda i: (i, 0)),
        ],
        out_specs=[
            pl.BlockSpec((blk, BH), lambda i: (i, 0)),
            pl.BlockSpec((blk, BH), lambda i: (i, 0)),
        ],
        out_shape=[
            jax.ShapeDtypeStruct((N_B, BH), jnp.float32),
            jax.ShapeDtypeStruct((N_B, BH), jnp.float32),
        ],
        compiler_params=pltpu.CompilerParams(
            dimension_semantics=("parallel",)),
    )(W_agg, x_t, b_agg.reshape(N_B, 1))

    agg0 = _sc_spmm(hT0, rows2, cols2, vals2)
    agg1 = _sc_spmm(hT1, rows2, cols2, vals2)

    cblk = 1280
    cgrid = (N_B + cblk - 1) // cblk
    out = pl.pallas_call(
        _out_body,
        grid=(cgrid,),
        in_specs=[
            pl.BlockSpec((B, D), lambda j: (0, 0)),
            pl.BlockSpec((cblk, D), lambda j: (j, 0)),
            pl.BlockSpec((1, cblk), lambda j: (0, j)),
            pl.BlockSpec((cblk, BH), lambda j: (j, 0)),
            pl.BlockSpec((cblk, BH), lambda j: (j, 0)),
        ],
        out_specs=pl.BlockSpec((B, cblk), lambda j: (0, j)),
        out_shape=jax.ShapeDtypeStruct((B, N_B), jnp.float32),
        compiler_params=pltpu.CompilerParams(
            dimension_semantics=("parallel",)),
    )(input_feature, W1, b1.reshape(1, N_B), agg0, agg1)
    return out


# speedup vs baseline: 2.5234x; 1.0476x over previous
import functools

import jax
import jax.numpy as jnp
from jax import lax
from jax.experimental import pallas as pl
from jax.experimental.pallas import tpu as pltpu
from jax.experimental.pallas import tpu_sc as plsc

N_B = 10000
NNZ = 320000
B = 256
BH = B // 2
D = 768
RA_SCALE = 2.0

NCORE = 2
NSUB = 16
ROWS_PER_CORE = N_B // NCORE
ACC_ROWS = 5008
DUMMY = ROWS_PER_CORE
CH = 80
CPT = NNZ // NSUB // CH
SBC = 50
NSB = CPT // SBC
ZR = 64
ROWS_PER_TILE = 312


def _h_body(w_ref, xt_ref, b_ref, o0_ref, o1_ref):
    h = (jnp.dot(w_ref[...], xt_ref[...], preferred_element_type=jnp.float32)
         + b_ref[...])
    o0_ref[...] = h[:, :BH]
    o1_ref[...] = h[:, BH:]


def _out_body(x_ref, w_ref, b_ref, a0_ref, a1_ref, o_ref):
    mm = lax.dot_general(
        x_ref[...], w_ref[...], (((1,), (1,)), ((), ())),
        preferred_element_type=jnp.float32,
    )
    aT = jnp.concatenate([a0_ref[...].T, a1_ref[...].T], axis=0)
    o_ref[...] = mm + b_ref[...] + RA_SCALE * aT


_sc_mesh = plsc.VectorSubcoreMesh(core_axis_name="c", subcore_axis_name="s")


@functools.partial(
    pl.kernel,
    out_type=jax.ShapeDtypeStruct((N_B, BH), jnp.float32),
    mesh=_sc_mesh,
    compiler_params=pltpu.CompilerParams(needs_layout_passes=False),
    scratch_types=[
        pltpu.VMEM((SBC, 1, CH), jnp.int32),
        pltpu.VMEM((SBC, 1, CH), jnp.int32),
        pltpu.VMEM((SBC * CH,), jnp.float32),
        pltpu.VMEM((SBC, 1, CH), jnp.int32),
        pltpu.VMEM((CH, BH), jnp.float32),
        pltpu.VMEM((CH, BH), jnp.float32),
        pltpu.VMEM((CH, BH), jnp.float32),
        pltpu.VMEM((CH, BH), jnp.float32),
        pltpu.VMEM((ZR, BH), jnp.float32),
        pltpu.VMEM_SHARED((ACC_ROWS, BH), jnp.float32),
        pltpu.SemaphoreType.DMA,
        pltpu.SemaphoreType.DMA,
        pltpu.SemaphoreType.DMA,
        pltpu.SemaphoreType.DMA,
        pltpu.SemaphoreType.DMA,
        pltpu.SemaphoreType.DMA,
        pltpu.SemaphoreType.DMA,
        pltpu.SemaphoreType.DMA,
    ],
)
def _sc_spmm(hT, rows2, cols2, vals2, out,
             rbuf, cbuf, vbuf, ibuf, gbuf0, gbuf1, gbuf2, gbuf3, zbuf, acc,
             gsem0, gsem1, gsem2, gsem3, ssem0, ssem1, ssem2, ssem3):
    c = lax.axis_index("c")
    s = lax.axis_index("s")
    core_base = c * ROWS_PER_CORE
    gbufs = (gbuf0, gbuf1, gbuf2, gbuf3)
    gsems = (gsem0, gsem1, gsem2, gsem3)
    ssems = (ssem0, ssem1, ssem2, ssem3)

    def _zrow(r, carry):
        for t in range(BH // 16):
            zbuf[r, pl.ds(t * 16, 16)] = jnp.zeros((16,), jnp.float32)
        return carry

    lax.fori_loop(0, ZR, _zrow, 0)
    zoff = 0
    for zlen in (ZR, ZR, ZR, ZR, ROWS_PER_TILE - 4 * ZR):
        pltpu.sync_copy(
            zbuf.at[pl.ds(0, zlen)],
            acc.at[pl.ds(s * ROWS_PER_TILE + zoff, zlen)])
        zoff += zlen

    @pl.when(s == 0)
    def _ztail():
        pltpu.sync_copy(
            zbuf.at[pl.ds(0, ACC_ROWS - NSUB * ROWS_PER_TILE)],
            acc.at[pl.ds(NSUB * ROWS_PER_TILE,
                         ACC_ROWS - NSUB * ROWS_PER_TILE)])

    plsc.subcore_barrier()

    def _scale_chunk(buf, j):
        jc = j * CH

        def _sgrp(g, carry):
            v16 = vbuf[pl.ds(jc + g * 16, 16)]
            for jj in range(16):
                vv = jnp.take_along_axis(
                    v16, jnp.full((16,), jj, jnp.int32), axis=0)
                k = g * 16 + jj
                for t in range(BH // 16):
                    buf[k, pl.ds(t * 16, 16)] = buf[k, pl.ds(t * 16, 16)] * vv
            return carry

        lax.fori_loop(0, CH // 16, _sgrp, 0)

    def _superblock(sb, carry):
        cb = s * CPT + sb * SBC
        pltpu.sync_copy(rows2.at[pl.ds(cb, SBC)], rbuf)
        pltpu.sync_copy(cols2.at[pl.ds(cb, SBC)], cbuf)
        pltpu.sync_copy(vals2.at[pl.ds(cb * CH, SBC * CH)], vbuf)

        def _idx(j, icarry):
            for v in range(CH // 16):
                r = rbuf[j, 0, pl.ds(v * 16, 16)]
                lr = r - core_base
                m = (lr >= 0) & (lr < ROWS_PER_CORE)
                ibuf[j, 0, pl.ds(v * 16, 16)] = jnp.where(m, lr, DUMMY)
            return icarry

        lax.fori_loop(0, SBC, _idx, 0)

        def _quad(qq, icarry):
            jo = qq * 4
            for b in range(4):
                @pl.when((qq > 0) | (sb > 0))
                def _drain(b=b, jo=jo):
                    pltpu.make_async_copy(
                        gbufs[b], acc.at[ibuf.at[jo, 0]], ssems[b]).wait()

                pltpu.async_copy(hT.at[cbuf.at[jo + b, 0]], gbufs[b], gsems[b])
            for b in range(4):
                pltpu.make_async_copy(
                    hT.at[cbuf.at[jo + b, 0]], gbufs[b], gsems[b]).wait()
                _scale_chunk(gbufs[b], jo + b)
                pltpu.async_copy(
                    gbufs[b], acc.at[ibuf.at[jo + b, 0]], ssems[b], add=True)
            return icarry

        lax.fori_loop(0, SBC // 4, _quad, 0)

        jt = (SBC // 4) * 4
        pltpu.make_async_copy(gbuf0, acc.at[ibuf.at[jt, 0]], ssem0).wait()
        pltpu.make_async_copy(gbuf1, acc.at[ibuf.at[jt, 0]], ssem1).wait()
        g0 = pltpu.async_copy(hT.at[cbuf.at[jt, 0]], gbuf0, gsem0)
        g1 = pltpu.async_copy(hT.at[cbuf.at[jt + 1, 0]], gbuf1, gsem1)
        g0.wait()
        _scale_chunk(gbuf0, jt)
        pltpu.async_copy(gbuf0, acc.at[ibuf.at[jt, 0]], ssem0, add=True)
        g1.wait()
        _scale_chunk(gbuf1, jt + 1)
        pltpu.async_copy(gbuf1, acc.at[ibuf.at[jt + 1, 0]], ssem1, add=True)
        return carry

    lax.fori_loop(0, NSB, _superblock, 0)

    pltpu.make_async_copy(gbuf0, acc.at[ibuf.at[0, 0]], ssem0).wait()
    pltpu.make_async_copy(gbuf1, acc.at[ibuf.at[0, 0]], ssem1).wait()
    pltpu.make_async_copy(gbuf2, acc.at[ibuf.at[0, 0]], ssem2).wait()
    pltpu.make_async_copy(gbuf3, acc.at[ibuf.at[0, 0]], ssem3).wait()
    plsc.subcore_barrier()

    base = s * ROWS_PER_TILE
    pltpu.sync_copy(
        acc.at[pl.ds(base, ROWS_PER_TILE)],
        out.at[pl.ds(c * ROWS_PER_CORE + base, ROWS_PER_TILE)],
    )

    @pl.when(s == 0)
    def _wtail():
        pltpu.sync_copy(
            acc.at[pl.ds(NSUB * ROWS_PER_TILE,
                         ROWS_PER_CORE - NSUB * ROWS_PER_TILE)],
            out.at[pl.ds(c * ROWS_PER_CORE + NSUB * ROWS_PER_TILE,
                         ROWS_PER_CORE - NSUB * ROWS_PER_TILE)],
        )


def kernel(input_feature, adj_indices, adj_values, W1, b1, W_agg, b_agg):
    x_t = input_feature.T
    rows2 = adj_indices[0].reshape(NNZ // CH, 1, CH)
    cols2 = adj_indices[1].reshape(NNZ // CH, 1, CH)
    vals2 = adj_values

    blk = 1000
    grid = N_B // blk

    hT0, hT1 = pl.pallas_call(
        _h_body,
        grid=(grid,),
        in_specs=[
            pl.BlockSpec((blk, D), lambda i: (i, 0)),
            pl.BlockSpec((D, B), lambda i: (0, 0)),
            pl.BlockSpec((blk, 1), lambda i: (i, 0)),
        ],
        out_specs=[
            pl.BlockSpec((blk, BH), lambda i: (i, 0)),
            pl.BlockSpec((blk, BH), lambda i: (i, 0)),
        ],
        out_shape=[
            jax.ShapeDtypeStruct((N_B, BH), jnp.float32),
            jax.ShapeDtypeStruct((N_B, BH), jnp.float32),
        ],
        compiler_params=pltpu.CompilerParams(
            dimension_semantics=("parallel",)),
    )(W_agg, x_t, b_agg.reshape(N_B, 1))

    agg0 = _sc_spmm(hT0, rows2, cols2, vals2)
    agg1 = _sc_spmm(hT1, rows2, cols2, vals2)

    cblk = 1280
    cgrid = (N_B + cblk - 1) // cblk
    out = pl.pallas_call(
        _out_body,
        grid=(cgrid,),
        in_specs=[
            pl.BlockSpec((B, D), lambda j: (0, 0)),
            pl.BlockSpec((cblk, D), lambda j: (j, 0)),
            pl.BlockSpec((1, cblk), lambda j: (0, j)),
            pl.BlockSpec((cblk, BH), lambda j: (j, 0)),
            pl.BlockSpec((cblk, BH), lambda j: (j, 0)),
        ],
        out_specs=pl.BlockSpec((B, cblk), lambda j: (0, j)),
        out_shape=jax.ShapeDtypeStruct((B, N_B), jnp.float32),
        compiler_params=pltpu.CompilerParams(
            dimension_semantics=("parallel",)),
    )(input_feature, W1, b1.reshape(1, N_B), agg0, agg1)
    return out
